# baseline (speedup 1.0000x reference)
import jax
import jax.numpy as jnp
from jax import lax
from jax.experimental import pallas as pl
from jax.experimental.pallas import tpu as pltpu
from jax.experimental.pallas import tpu_sc as plsc

_N = 10000
_E = 320000
_D = 128
_NC = 2
_NS = 16
_NW = _NC * _NS
_EPW = _E // _NW
_K = 40
_NCHUNK = _EPW // _K
_R = 5
_RPT0 = 632
_RPTL = _N - (_NS - 1) * _RPT0


def _linear_body(x_ref, w_ref, b_ref, o_ref):
    o_ref[...] = lax.dot_general(
        x_ref[...], w_ref[...], (((1,), (1,)), ((), ())),
        preferred_element_type=jnp.float32) + b_ref[...]


def _linear(x, W, b):
    bm = 1000
    return pl.pallas_call(
        _linear_body,
        grid=(_N // bm,),
        in_specs=[
            pl.BlockSpec((bm, _D), lambda i: (i, 0)),
            pl.BlockSpec((_D, _D), lambda i: (0, 0)),
            pl.BlockSpec((1, _D), lambda i: (0, 0)),
        ],
        out_specs=pl.BlockSpec((bm, _D), lambda i: (i, 0)),
        out_shape=jax.ShapeDtypeStruct((_N, _D), jnp.float32),
    )(x, W, b.reshape(1, _D))


def _splat_lane(vec16, lane):
    return lax.gather(
        vec16, jnp.full((16, 1), lane, jnp.int32),
        lax.GatherDimensionNumbers(
            offset_dims=(), collapsed_slice_dims=(0,), start_index_map=(0,)),
        slice_sizes=(1,),
        mode=lax.GatherScatterMode.PROMISE_IN_BOUNDS)


def _sc_agg_body(y_hbm, pk_hbm, out_hbm,
                 pbuf, rbuf, gbuf, acc, gsem, psem, ssem):
    c = lax.axis_index("c")
    s = lax.axis_index("s")
    wid = s * _NC + c

    def _zrow(r, carry):
        for j in range(_D // 16):
            gbuf[0, r, pl.ds(j * 16, 16)] = jnp.zeros((16,), jnp.float32)
        return carry
    lax.fori_loop(0, _K, _zrow, 0)

    @pl.when(s < _NS - 1)
    def _():
        for q in range(_RPT0 // _K):
            pltpu.sync_copy(gbuf.at[0],
                            acc.at[pl.ds(s * _RPT0 + q * _K, _K)])
        rem = _RPT0 % _K
        if rem:
            pltpu.sync_copy(gbuf.at[0, pl.ds(0, rem)],
                            acc.at[pl.ds(s * _RPT0 + _RPT0 - rem, rem)])

    @pl.when(s == _NS - 1)
    def _():
        for q in range(_RPTL // _K):
            pltpu.sync_copy(gbuf.at[0],
                            acc.at[pl.ds(s * _RPT0 + q * _K, _K)])

    for j in range(_R):
        pltpu.async_copy(pk_hbm.at[wid, j], pbuf.at[j], psem.at[j])
    plsc.subcore_barrier()
    for j in range(3):
        pltpu.make_async_copy(pk_hbm.at[wid, j], pbuf.at[j],
                              psem.at[j]).wait()
        pltpu.async_copy(y_hbm.at[pbuf.at[j, 0]], gbuf.at[j], gsem.at[j])

    def _scale_16(b, val16, row0, lanes):
        for e in lanes:
            vsplat = _splat_lane(val16, e)
            row = row0 + e
            for j in range(_D // 16):
                slj = pl.ds(j * 16, 16)
                gbuf[b, row, slj] = gbuf[b, row, slj] * vsplat

    def _multiply(b):
        def _grp(p, carry):
            sl16 = pl.ds(p * 16, 16)
            rbuf[b, sl16] = pbuf[b, 1, sl16]
            val16 = lax.bitcast_convert_type(pbuf[b, 2, sl16], jnp.float32)
            _scale_16(b, val16, p * 16, range(16))
            return carry
        lax.fori_loop(0, 2, _grp, 0)
        sl16 = pl.ds(24, 16)
        rbuf[b, sl16] = pbuf[b, 1, sl16]
        val16 = lax.bitcast_convert_type(pbuf[b, 2, sl16], jnp.float32)
        _scale_16(b, val16, 24, range(8, 16))

    def _chunk(ci, carry):
        b = lax.rem(ci, _R)
        b3 = lax.rem(ci + 3, _R)
        pltpu.make_async_copy(y_hbm.at[pbuf.at[b, 0]], gbuf.at[b],
                              gsem.at[b]).wait()
        _multiply(b)
        pltpu.async_copy(gbuf.at[b], acc.at[rbuf.at[b]], ssem.at[b],
                         add=True)

        @pl.when(ci + _R <= _NCHUNK - 1)
        def _():
            pltpu.async_copy(pk_hbm.at[wid, ci + _R], pbuf.at[b],
                             psem.at[b])

        @pl.when(ci >= 2)
        def _():
            pltpu.make_async_copy(gbuf.at[b3], acc.at[rbuf.at[b3]],
                                  ssem.at[b3]).wait()

        @pl.when(ci + 3 <= _NCHUNK - 1)
        def _():
            pltpu.make_async_copy(pk_hbm.at[wid, ci + 3], pbuf.at[b3],
                                  psem.at[b3]).wait()
            pltpu.async_copy(y_hbm.at[pbuf.at[b3, 0]], gbuf.at[b3],
                             gsem.at[b3])
        return carry
    lax.fori_loop(0, _NCHUNK, _chunk, 0)

    for j in (3, 4):
        pltpu.make_async_copy(gbuf.at[j], acc.at[rbuf.at[j]],
                              ssem.at[j]).wait()

    plsc.subcore_barrier()
    base = c * _N + s * _RPT0

    @pl.when(s < _NS - 1)
    def _():
        pltpu.sync_copy(acc.at[pl.ds(s * _RPT0, _RPT0)],
                        out_hbm.at[pl.ds(base, _RPT0)])

    @pl.when(s == _NS - 1)
    def _():
        pltpu.sync_copy(acc.at[pl.ds(s * _RPT0, _RPTL)],
                        out_hbm.at[pl.ds(base, _RPTL)])


def _sc_agg(y, pk):
    mesh = plsc.VectorSubcoreMesh(core_axis_name="c", subcore_axis_name="s")
    fn = pl.kernel(
        _sc_agg_body,
        mesh=mesh,
        out_type=jax.ShapeDtypeStruct((_NC * _N, _D), jnp.float32),
        scratch_types=[
            pltpu.VMEM((_R, 3, _K), jnp.int32),
            pltpu.VMEM((_R, _K), jnp.int32),
            pltpu.VMEM((_R, _K, _D), jnp.float32),
            pltpu.VMEM_SHARED((_N, _D), jnp.float32),
            pltpu.SemaphoreType.DMA((_R,)),
            pltpu.SemaphoreType.DMA((_R,)),
            pltpu.SemaphoreType.DMA((_R,)),
        ],
    )
    return fn(y, pk)


def _combine_body(a_ref, b_ref, o_ref):
    o_ref[...] = a_ref[...] + b_ref[...]


def _combine(partials):
    bm = 1000
    nb = _N // bm
    return pl.pallas_call(
        _combine_body,
        grid=(nb,),
        in_specs=[
            pl.BlockSpec((bm, _D), lambda i: (i, 0)),
            pl.BlockSpec((bm, _D), lambda i: (i + nb, 0)),
        ],
        out_specs=pl.BlockSpec((bm, _D), lambda i: (i, 0)),
        out_shape=jax.ShapeDtypeStruct((_N, _D), jnp.float32),
    )(partials, partials)


def kernel(x, adj_indices, adj_values, W, b):
    row = adj_indices[0].astype(jnp.int32).reshape(_NW, _NCHUNK, _K)
    col = adj_indices[1].astype(jnp.int32).reshape(_NW, _NCHUNK, _K)
    valbits = lax.bitcast_convert_type(
        adj_values.astype(jnp.float32), jnp.int32).reshape(_NW, _NCHUNK, _K)
    pk = jnp.stack([col, row, valbits], axis=2)
    y = _linear(x, W, b)
    partials = _sc_agg(y, pk)
    return _combine(partials)


# speedup vs baseline: 1.9293x; 1.9293x over previous
import jax
import jax.numpy as jnp
from jax import lax
from jax.experimental import pallas as pl
from jax.experimental.pallas import tpu as pltpu
from jax.experimental.pallas import tpu_sc as plsc

_N = 10000
_E = 320000
_D = 128
_NC = 2
_NS = 16
_NW = _NC * _NS
_EPW = _E // _NW
_K = 40
_NCHUNK = _EPW // _K
_R = 5
_RPT0 = 632
_RPTL = _N - (_NS - 1) * _RPT0


def _linear_body(x_ref, w_ref, b_ref, o_ref):
    o_ref[...] = lax.dot_general(
        x_ref[...], w_ref[...], (((1,), (1,)), ((), ())),
        preferred_element_type=jnp.float32) + b_ref[...]


def _linear(x, W, b):
    bm = 1000
    return pl.pallas_call(
        _linear_body,
        grid=(_N // bm,),
        in_specs=[
            pl.BlockSpec((bm, _D), lambda i: (i, 0)),
            pl.BlockSpec((_D, _D), lambda i: (0, 0)),
            pl.BlockSpec((1, _D), lambda i: (0, 0)),
        ],
        out_specs=pl.BlockSpec((bm, _D), lambda i: (i, 0)),
        out_shape=jax.ShapeDtypeStruct((_N, _D), jnp.float32),
    )(x, W, b.reshape(1, _D))


def _splat_lane(vec16, lane):
    return lax.gather(
        vec16, jnp.full((16, 1), lane, jnp.int32),
        lax.GatherDimensionNumbers(
            offset_dims=(), collapsed_slice_dims=(0,), start_index_map=(0,)),
        slice_sizes=(1,),
        mode=lax.GatherScatterMode.PROMISE_IN_BOUNDS)


def _sc_agg_body(y_hbm, pk_hbm, out_hbm,
                 pbuf, rbuf, gbuf, acc, gsem, psem, ssem):
    c = lax.axis_index("c")
    s = lax.axis_index("s")
    wid = s * _NC + c

    def _zrow(r, carry):
        for j in range(_D // 16):
            gbuf[0, r, pl.ds(j * 16, 16)] = jnp.zeros((16,), jnp.float32)
        return carry
    lax.fori_loop(0, _K, _zrow, 0)

    @pl.when(s < _NS - 1)
    def _():
        for q in range(_RPT0 // _K):
            pltpu.sync_copy(gbuf.at[0],
                            acc.at[pl.ds(s * _RPT0 + q * _K, _K)])
        rem = _RPT0 % _K
        if rem:
            pltpu.sync_copy(gbuf.at[0, pl.ds(0, rem)],
                            acc.at[pl.ds(s * _RPT0 + _RPT0 - rem, rem)])

    @pl.when(s == _NS - 1)
    def _():
        for q in range(_RPTL // _K):
            pltpu.sync_copy(gbuf.at[0],
                            acc.at[pl.ds(s * _RPT0 + q * _K, _K)])

    for j in range(_R):
        pltpu.async_copy(pk_hbm.at[wid, j], pbuf.at[j], psem.at[j])
    plsc.subcore_barrier()
    for j in range(3):
        pltpu.make_async_copy(pk_hbm.at[wid, j], pbuf.at[j],
                              psem.at[j]).wait()
        pltpu.async_copy(y_hbm.at[pbuf.at[j, 0]], gbuf.at[j], gsem.at[j])

    def _scale_16(b, val16, row0, lanes):
        for e in lanes:
            vsplat = _splat_lane(val16, e)
            row = row0 + e
            for j in range(_D // 16):
                slj = pl.ds(j * 16, 16)
                gbuf[b, row, slj] = gbuf[b, row, slj] * vsplat

    def _multiply(b):
        def _grp(p, carry):
            sl16 = pl.ds(p * 16, 16)
            rbuf[b, sl16] = pbuf[b, 1, sl16]
            val16 = lax.bitcast_convert_type(pbuf[b, 2, sl16], jnp.float32)
            _scale_16(b, val16, p * 16, range(16))
            return carry
        lax.fori_loop(0, 2, _grp, 0)
        sl16 = pl.ds(24, 16)
        rbuf[b, sl16] = pbuf[b, 1, sl16]
        val16 = lax.bitcast_convert_type(pbuf[b, 2, sl16], jnp.float32)
        _scale_16(b, val16, 24, range(8, 16))

    def _chunk(ci, b, k):
        b3 = (b + 3) % _R
        pltpu.make_async_copy(y_hbm.at[pbuf.at[b, 0]], gbuf.at[b],
                              gsem.at[b]).wait()
        _multiply(b)
        pltpu.async_copy(gbuf.at[b], acc.at[rbuf.at[b]], ssem.at[b],
                         add=True)

        def _prefetch():
            pltpu.async_copy(pk_hbm.at[wid, ci + _R], pbuf.at[b],
                             psem.at[b])
        pl.when(k < _NCHUNK // _R - 1)(_prefetch)

        def _wait_s():
            pltpu.make_async_copy(gbuf.at[b3], acc.at[rbuf.at[b3]],
                                  ssem.at[b3]).wait()
        if b < 2:
            pl.when(k >= 1)(_wait_s)
        else:
            _wait_s()

        def _next_gather():
            pltpu.make_async_copy(pk_hbm.at[wid, ci + 3], pbuf.at[b3],
                                  psem.at[b3]).wait()
            pltpu.async_copy(y_hbm.at[pbuf.at[b3, 0]], gbuf.at[b3],
                             gsem.at[b3])
        if b < 2:
            _next_gather()
        else:
            pl.when(k < _NCHUNK // _R - 1)(_next_gather)

    def _kbody(k, carry):
        base = _R * k
        for u in range(_R):
            _chunk(base + u, u, k)
        return carry
    lax.fori_loop(0, _NCHUNK // _R, _kbody, 0)

    for j in (3, 4):
        pltpu.make_async_copy(gbuf.at[j], acc.at[rbuf.at[j]],
                              ssem.at[j]).wait()

    plsc.subcore_barrier()
    base = c * _N + s * _RPT0

    @pl.when(s < _NS - 1)
    def _():
        pltpu.sync_copy(acc.at[pl.ds(s * _RPT0, _RPT0)],
                        out_hbm.at[pl.ds(base, _RPT0)])

    @pl.when(s == _NS - 1)
    def _():
        pltpu.sync_copy(acc.at[pl.ds(s * _RPT0, _RPTL)],
                        out_hbm.at[pl.ds(base, _RPTL)])


def _sc_agg(y, pk):
    mesh = plsc.VectorSubcoreMesh(core_axis_name="c", subcore_axis_name="s")
    fn = pl.kernel(
        _sc_agg_body,
        mesh=mesh,
        out_type=jax.ShapeDtypeStruct((_NC * _N, _D), jnp.float32),
        scratch_types=[
            pltpu.VMEM((_R, 3, _K), jnp.int32),
            pltpu.VMEM((_R, _K), jnp.int32),
            pltpu.VMEM((_R, _K, _D), jnp.float32),
            pltpu.VMEM_SHARED((_N, _D), jnp.float32),
            pltpu.SemaphoreType.DMA((_R,)),
            pltpu.SemaphoreType.DMA((_R,)),
            pltpu.SemaphoreType.DMA((_R,)),
        ],
    )
    return fn(y, pk)


def _combine_body(a_ref, b_ref, o_ref):
    o_ref[...] = a_ref[...] + b_ref[...]


def _combine(partials):
    bm = 1000
    nb = _N // bm
    return pl.pallas_call(
        _combine_body,
        grid=(nb,),
        in_specs=[
            pl.BlockSpec((bm, _D), lambda i: (i, 0)),
            pl.BlockSpec((bm, _D), lambda i: (i + nb, 0)),
        ],
        out_specs=pl.BlockSpec((bm, _D), lambda i: (i, 0)),
        out_shape=jax.ShapeDtypeStruct((_N, _D), jnp.float32),
    )(partials, partials)


def kernel(x, adj_indices, adj_values, W, b):
    row = adj_indices[0].astype(jnp.int32).reshape(_NW, _NCHUNK, _K)
    col = adj_indices[1].astype(jnp.int32).reshape(_NW, _NCHUNK, _K)
    valbits = lax.bitcast_convert_type(
        adj_values.astype(jnp.float32), jnp.int32).reshape(_NW, _NCHUNK, _K)
    pk = jnp.stack([col, row, valbits], axis=2)
    y = _linear(x, W, b)
    partials = _sc_agg(y, pk)
    return _combine(partials)


# speedup vs baseline: 2.1396x; 1.1090x over previous
import jax
import jax.numpy as jnp
from jax import lax
from jax.experimental import pallas as pl
from jax.experimental.pallas import tpu as pltpu
from jax.experimental.pallas import tpu_sc as plsc

_N = 10000
_E = 320000
_D = 128
_NC = 2
_NS = 16
_NW = _NC * _NS
_EPW = _E // _NW
_K = 80
_KH = _K // 2
_NCHUNK = _EPW // _K
_R = 3
_RPT0 = 632
_RPTL = _N - (_NS - 1) * _RPT0


def _linear_body(x_ref, w_ref, b_ref, o_ref):
    o_ref[...] = lax.dot_general(
        x_ref[...], w_ref[...], (((1,), (1,)), ((), ())),
        preferred_element_type=jnp.float32) + b_ref[...]


def _linear(x, W, b):
    bm = 1000
    return pl.pallas_call(
        _linear_body,
        grid=(_N // bm,),
        in_specs=[
            pl.BlockSpec((bm, _D), lambda i: (i, 0)),
            pl.BlockSpec((_D, _D), lambda i: (0, 0)),
            pl.BlockSpec((1, _D), lambda i: (0, 0)),
        ],
        out_specs=pl.BlockSpec((bm, _D), lambda i: (i, 0)),
        out_shape=jax.ShapeDtypeStruct((_N, _D), jnp.float32),
    )(x, W, b.reshape(1, _D))


def _splat_lane(vec16, lane):
    return lax.gather(
        vec16, jnp.full((16, 1), lane, jnp.int32),
        lax.GatherDimensionNumbers(
            offset_dims=(), collapsed_slice_dims=(0,), start_index_map=(0,)),
        slice_sizes=(1,),
        mode=lax.GatherScatterMode.PROMISE_IN_BOUNDS)


def _sc_agg_body(y_hbm, pk_hbm, out_hbm,
                 pbuf, rbuf, gbuf, acc, gsemA, gsemB, psem, ssem):
    c = lax.axis_index("c")
    s = lax.axis_index("s")
    wid = s * _NC + c

    def _zrow(r, carry):
        for j in range(_D // 16):
            gbuf[0, r, pl.ds(j * 16, 16)] = jnp.zeros((16,), jnp.float32)
        return carry
    lax.fori_loop(0, _K, _zrow, 0)

    @pl.when(s < _NS - 1)
    def _():
        for q in range(_RPT0 // _K):
            pltpu.sync_copy(gbuf.at[0],
                            acc.at[pl.ds(s * _RPT0 + q * _K, _K)])
        rem = _RPT0 % _K
        if rem:
            pltpu.sync_copy(gbuf.at[0, pl.ds(0, rem)],
                            acc.at[pl.ds(s * _RPT0 + _RPT0 - rem, rem)])

    @pl.when(s == _NS - 1)
    def _():
        for q in range(_RPTL // _K):
            pltpu.sync_copy(gbuf.at[0],
                            acc.at[pl.ds(s * _RPT0 + q * _K, _K)])
        rem = _RPTL % _K
        if rem:
            pltpu.sync_copy(gbuf.at[0, pl.ds(0, rem)],
                            acc.at[pl.ds(s * _RPT0 + _RPTL - rem, rem)])

    def _gather_half(b, h, sem_slot):
        sem = gsemA if h == 0 else gsemB
        return pltpu.make_async_copy(
            y_hbm.at[pbuf.at[b, 0, pl.ds(h * _KH, _KH)]],
            gbuf.at[b, pl.ds(h * _KH, _KH)],
            sem.at[sem_slot])

    def _start_gather(b):
        _gather_half(b, 0, b).start()
        _gather_half(b, 1, b).start()

    def _wait_gather(b):
        _gather_half(b, 0, b).wait()
        _gather_half(b, 1, b).wait()

    for j in range(_R):
        pltpu.async_copy(pk_hbm.at[wid, j], pbuf.at[j], psem.at[j])
    plsc.subcore_barrier()
    for j in range(2):
        pltpu.make_async_copy(pk_hbm.at[wid, j], pbuf.at[j],
                              psem.at[j]).wait()
        _start_gather(j)

    def _multiply(b):
        def _grp(g, carry):
            sl16 = pl.ds(g * 16, 16)
            rbuf[b, sl16] = pbuf[b, 1, sl16]
            val16 = lax.bitcast_convert_type(pbuf[b, 2, sl16], jnp.float32)
            for e in range(16):
                vsplat = _splat_lane(val16, e)
                row = g * 16 + e
                for j in range(_D // 16):
                    slj = pl.ds(j * 16, 16)
                    gbuf[b, row, slj] = gbuf[b, row, slj] * vsplat
            return carry
        lax.fori_loop(0, _K // 16, _grp, 0)

    def _chunk(ci, b, k=None, depth2=True):
        b2 = (b + 2) % _R
        _wait_gather(b)
        _multiply(b)
        pltpu.async_copy(gbuf.at[b], acc.at[rbuf.at[b]], ssem.at[b],
                         add=True)
        if depth2:
            pltpu.async_copy(
                pk_hbm.at[wid, jnp.minimum(ci + 3, _NCHUNK - 1)],
                pbuf.at[b], psem.at[b])
        def _wait_s():
            pltpu.make_async_copy(gbuf.at[b2], acc.at[rbuf.at[b2]],
                                  ssem.at[b2]).wait()
        if k is None:
            _wait_s()
        else:
            pl.when(k >= 1)(_wait_s)
        if depth2:
            pltpu.make_async_copy(pk_hbm.at[wid, ci + 2], pbuf.at[b2],
                                  psem.at[b2]).wait()
            _start_gather(b2)

    def _kbody(k, carry):
        base = 3 * k
        _chunk(base + 0, 0, k=k)
        _chunk(base + 1, 1)
        _chunk(base + 2, 2)
        return carry
    lax.fori_loop(0, 41, _kbody, 0)

    _chunk(_NCHUNK - 2, 0, depth2=False)
    _chunk(_NCHUNK - 1, 1, depth2=False)
    pltpu.make_async_copy(gbuf.at[1], acc.at[rbuf.at[1]], ssem.at[1]).wait()
    pltpu.make_async_copy(pk_hbm.at[wid, 0], pbuf.at[2], psem.at[2]).wait()

    plsc.subcore_barrier()
    base = c * _N + s * _RPT0

    @pl.when(s < _NS - 1)
    def _():
        pltpu.sync_copy(acc.at[pl.ds(s * _RPT0, _RPT0)],
                        out_hbm.at[pl.ds(base, _RPT0)])

    @pl.when(s == _NS - 1)
    def _():
        pltpu.sync_copy(acc.at[pl.ds(s * _RPT0, _RPTL)],
                        out_hbm.at[pl.ds(base, _RPTL)])


def _sc_agg(y, pk):
    mesh = plsc.VectorSubcoreMesh(core_axis_name="c", subcore_axis_name="s")
    fn = pl.kernel(
        _sc_agg_body,
        mesh=mesh,
        out_type=jax.ShapeDtypeStruct((_NC * _N, _D), jnp.float32),
        scratch_types=[
            pltpu.VMEM((_R, 3, _K), jnp.int32),
            pltpu.VMEM((_R, _K), jnp.int32),
            pltpu.VMEM((_R, _K, _D), jnp.float32),
            pltpu.VMEM_SHARED((_N, _D), jnp.float32),
            pltpu.SemaphoreType.DMA((_R,)),
            pltpu.SemaphoreType.DMA((_R,)),
            pltpu.SemaphoreType.DMA((_R,)),
            pltpu.SemaphoreType.DMA((_R,)),
        ],
    )
    return fn(y, pk)


def _combine_body(a_ref, b_ref, o_ref):
    o_ref[...] = a_ref[...] + b_ref[...]


def _combine(partials):
    bm = 1000
    nb = _N // bm
    return pl.pallas_call(
        _combine_body,
        grid=(nb,),
        in_specs=[
            pl.BlockSpec((bm, _D), lambda i: (i, 0)),
            pl.BlockSpec((bm, _D), lambda i: (i + nb, 0)),
        ],
        out_specs=pl.BlockSpec((bm, _D), lambda i: (i, 0)),
        out_shape=jax.ShapeDtypeStruct((_N, _D), jnp.float32),
    )(partials, partials)


def kernel(x, adj_indices, adj_values, W, b):
    row = adj_indices[0].astype(jnp.int32).reshape(_NW, _NCHUNK, _K)
    col = adj_indices[1].astype(jnp.int32).reshape(_NW, _NCHUNK, _K)
    valbits = lax.bitcast_convert_type(
        adj_values.astype(jnp.float32), jnp.int32).reshape(_NW, _NCHUNK, _K)
    pk = jnp.stack([col, row, valbits], axis=2)
    y = _linear(x, W, b)
    partials = _sc_agg(y, pk)
    return _combine(partials)


# speedup vs baseline: 2.5227x; 1.1791x over previous
import jax
import jax.numpy as jnp
from jax import lax
from jax.experimental import pallas as pl
from jax.experimental.pallas import tpu as pltpu
from jax.experimental.pallas import tpu_sc as plsc

_N = 10000
_E = 320000
_D = 128
_NC = 2
_NS = 16
_NW = _NC * _NS
_EPW = _E // _NW
_K = 80
_NCHUNK = _EPW // _K
_R = 3
_RPT0 = 632
_RPTL = _N - (_NS - 1) * _RPT0


def _linear_body(x_ref, w_ref, b_ref, o_ref):
    o_ref[...] = lax.dot_general(
        x_ref[...], w_ref[...], (((1,), (1,)), ((), ())),
        preferred_element_type=jnp.float32) + b_ref[...]


def _linear(x, W, b):
    bm = 1000
    return pl.pallas_call(
        _linear_body,
        grid=(_N // bm,),
        in_specs=[
            pl.BlockSpec((bm, _D), lambda i: (i, 0)),
            pl.BlockSpec((_D, _D), lambda i: (0, 0)),
            pl.BlockSpec((1, _D), lambda i: (0, 0)),
        ],
        out_specs=pl.BlockSpec((bm, _D), lambda i: (i, 0)),
        out_shape=jax.ShapeDtypeStruct((_N, _D), jnp.float32),
    )(x, W, b.reshape(1, _D))


def _splat_lane(vec16, lane):
    return lax.gather(
        vec16, jnp.full((16, 1), lane, jnp.int32),
        lax.GatherDimensionNumbers(
            offset_dims=(), collapsed_slice_dims=(0,), start_index_map=(0,)),
        slice_sizes=(1,),
        mode=lax.GatherScatterMode.PROMISE_IN_BOUNDS)


def _sc_agg_body(y_hbm, col_hbm, row_hbm, val_hbm, out_hbm,
                 cbuf, rbv, vbuf, rbuf, gbuf, acc, gsem, psem, ssem):
    c = lax.axis_index("c")
    s = lax.axis_index("s")
    wid = s * _NC + c
    ebase = wid * _EPW

    def _start_p(ci, slot):
        sl = pl.ds(ebase + ci * _K, _K)
        pltpu.async_copy(col_hbm.at[sl], cbuf.at[slot], psem.at[slot])
        pltpu.async_copy(row_hbm.at[sl], rbv.at[slot], psem.at[slot])
        pltpu.async_copy(val_hbm.at[sl], vbuf.at[slot], psem.at[slot])

    def _wait_p(ci, slot):
        sl = pl.ds(ebase + ci * _K, _K)
        pltpu.make_async_copy(col_hbm.at[sl], cbuf.at[slot],
                              psem.at[slot]).wait()
        pltpu.make_async_copy(row_hbm.at[sl], rbv.at[slot],
                              psem.at[slot]).wait()
        pltpu.make_async_copy(val_hbm.at[sl], vbuf.at[slot],
                              psem.at[slot]).wait()

    def _zrow(r, carry):
        for j in range(_D // 16):
            gbuf[2, r, pl.ds(j * 16, 16)] = jnp.zeros((16,), jnp.float32)
        return carry
    lax.fori_loop(0, _K, _zrow, 0)

    nz0, rem0 = _RPT0 // _K, _RPT0 % _K
    nzL, remL = _RPTL // _K, _RPTL % _K

    def _zero_copies(action):
        def _one(src, dst):
            d = pltpu.make_async_copy(src, dst, ssem.at[2])
            d.start() if action == "start" else d.wait()

        @pl.when(s < _NS - 1)
        def _():
            for q in range(nz0):
                _one(gbuf.at[2], acc.at[pl.ds(s * _RPT0 + q * _K, _K)])
            if rem0:
                _one(gbuf.at[2, pl.ds(0, rem0)],
                     acc.at[pl.ds(s * _RPT0 + _RPT0 - rem0, rem0)])

        @pl.when(s == _NS - 1)
        def _():
            for q in range(nzL):
                _one(gbuf.at[2], acc.at[pl.ds(s * _RPT0 + q * _K, _K)])
            if remL:
                _one(gbuf.at[2, pl.ds(0, remL)],
                     acc.at[pl.ds(s * _RPT0 + _RPTL - remL, remL)])

    _zero_copies("start")

    for j in range(_R):
        _start_p(j, j)
    for j in range(2):
        _wait_p(j, j)
        pltpu.async_copy(y_hbm.at[cbuf.at[j]], gbuf.at[j], gsem.at[j])

    _zero_copies("wait")
    plsc.subcore_barrier()

    def _multiply(b):
        def _grp(g, carry):
            sl16 = pl.ds(g * 16, 16)
            rbuf[b, sl16] = rbv[b, sl16]
            val16 = vbuf[b, sl16]
            for e in range(16):
                vsplat = _splat_lane(val16, e)
                row = g * 16 + e
                for j in range(_D // 16):
                    slj = pl.ds(j * 16, 16)
                    gbuf[b, row, slj] = gbuf[b, row, slj] * vsplat
            return carry
        lax.fori_loop(0, _K // 16, _grp, 0)

    def _chunk(ci, b, k=None, depth2=True):
        b2 = (b + 2) % _R
        pltpu.make_async_copy(y_hbm.at[cbuf.at[b]], gbuf.at[b],
                              gsem.at[b]).wait()
        _multiply(b)
        pltpu.async_copy(gbuf.at[b], acc.at[rbuf.at[b]], ssem.at[b],
                         add=True)
        if depth2:
            _start_p(jnp.minimum(ci + 3, _NCHUNK - 1), b)
        def _wait_s():
            pltpu.make_async_copy(gbuf.at[b2], acc.at[rbuf.at[b2]],
                                  ssem.at[b2]).wait()
        if k is None:
            _wait_s()
        else:
            pl.when(k >= 1)(_wait_s)
        if depth2:
            _wait_p(ci + 2, b2)
            pltpu.async_copy(y_hbm.at[cbuf.at[b2]], gbuf.at[b2],
                             gsem.at[b2])

    def _kbody(k, carry):
        base = 3 * k
        _chunk(base + 0, 0, k=k)
        _chunk(base + 1, 1)
        _chunk(base + 2, 2)
        return carry
    lax.fori_loop(0, 41, _kbody, 0)

    _chunk(_NCHUNK - 2, 0, depth2=False)
    _chunk(_NCHUNK - 1, 1, depth2=False)
    pltpu.make_async_copy(gbuf.at[1], acc.at[rbuf.at[1]], ssem.at[1]).wait()
    _wait_p(_NCHUNK - 1, 2)

    plsc.subcore_barrier()
    base = c * _N + s * _RPT0

    @pl.when(s < _NS - 1)
    def _():
        pltpu.sync_copy(acc.at[pl.ds(s * _RPT0, _RPT0)],
                        out_hbm.at[pl.ds(base, _RPT0)])

    @pl.when(s == _NS - 1)
    def _():
        pltpu.sync_copy(acc.at[pl.ds(s * _RPT0, _RPTL)],
                        out_hbm.at[pl.ds(base, _RPTL)])


def _sc_agg(y, col, row, val):
    mesh = plsc.VectorSubcoreMesh(core_axis_name="c", subcore_axis_name="s")
    fn = pl.kernel(
        _sc_agg_body,
        mesh=mesh,
        out_type=jax.ShapeDtypeStruct((_NC * _N, _D), jnp.float32),
        scratch_types=[
            pltpu.VMEM((_R, _K), jnp.int32),
            pltpu.VMEM((_R, _K), jnp.int32),
            pltpu.VMEM((_R, _K), jnp.float32),
            pltpu.VMEM((_R, _K), jnp.int32),
            pltpu.VMEM((_R, _K, _D), jnp.float32),
            pltpu.VMEM_SHARED((_N, _D), jnp.float32),
            pltpu.SemaphoreType.DMA((_R,)),
            pltpu.SemaphoreType.DMA((_R,)),
            pltpu.SemaphoreType.DMA((_R,)),
        ],
    )
    return fn(y, col, row, val)


def _combine_body(a_ref, b_ref, o_ref):
    o_ref[...] = a_ref[...] + b_ref[...]


def _combine(partials):
    bm = 1000
    nb = _N // bm
    return pl.pallas_call(
        _combine_body,
        grid=(nb,),
        in_specs=[
            pl.BlockSpec((bm, _D), lambda i: (i, 0)),
            pl.BlockSpec((bm, _D), lambda i: (i + nb, 0)),
        ],
        out_specs=pl.BlockSpec((bm, _D), lambda i: (i, 0)),
        out_shape=jax.ShapeDtypeStruct((_N, _D), jnp.float32),
    )(partials, partials)


def kernel(x, adj_indices, adj_values, W, b):
    row = adj_indices[0].astype(jnp.int32)
    col = adj_indices[1].astype(jnp.int32)
    val = adj_values.astype(jnp.float32)
    y = _linear(x, W, b)
    partials = _sc_agg(y, col, row, val)
    return _combine(partials)


# speedup vs baseline: 2.7707x; 1.0983x over previous
import jax
import jax.numpy as jnp
from jax import lax
from jax.experimental import pallas as pl
from jax.experimental.pallas import tpu as pltpu
from jax.experimental.pallas import tpu_sc as plsc

_N = 10000
_E = 320000
_D = 128
_NC = 2
_NS = 16
_NW = _NC * _NS
_EPW = _E // _NW
_K = 80
_NCHUNK = _EPW // _K
_R = 3
_RPT0 = 632
_RPTL = _N - (_NS - 1) * _RPT0


def _linear_body(x_ref, w_ref, b_ref, adj_ref, o_ref, row_ref, col_ref):
    o_ref[...] = lax.dot_general(
        x_ref[...], w_ref[...], (((1,), (1,)), ((), ())),
        preferred_element_type=jnp.float32) + b_ref[...]

    @pl.when(pl.program_id(0) == 0)
    def _():
        row_ref[...] = adj_ref[0, :]
        col_ref[...] = adj_ref[1, :]


def _linear(x, W, b, adj):
    bm = 1000
    return pl.pallas_call(
        _linear_body,
        grid=(_N // bm,),
        in_specs=[
            pl.BlockSpec((bm, _D), lambda i: (i, 0)),
            pl.BlockSpec((_D, _D), lambda i: (0, 0)),
            pl.BlockSpec((1, _D), lambda i: (0, 0)),
            pl.BlockSpec((2, _E), lambda i: (0, 0)),
        ],
        out_specs=[
            pl.BlockSpec((bm, _D), lambda i: (i, 0)),
            pl.BlockSpec((_E,), lambda i: (0,)),
            pl.BlockSpec((_E,), lambda i: (0,)),
        ],
        out_shape=[
            jax.ShapeDtypeStruct((_N, _D), jnp.float32),
            jax.ShapeDtypeStruct((_E,), jnp.int32),
            jax.ShapeDtypeStruct((_E,), jnp.int32),
        ],
    )(x, W, b.reshape(1, _D), adj)


def _splat_lane(vec16, lane):
    return lax.gather(
        vec16, jnp.full((16, 1), lane, jnp.int32),
        lax.GatherDimensionNumbers(
            offset_dims=(), collapsed_slice_dims=(0,), start_index_map=(0,)),
        slice_sizes=(1,),
        mode=lax.GatherScatterMode.PROMISE_IN_BOUNDS)


def _sc_agg_body(y_hbm, col_hbm, row_hbm, val_hbm, out_hbm,
                 cbuf, rbv, vbuf, rbuf, gbuf, acc, gsem, psem, ssem):
    c = lax.axis_index("c")
    s = lax.axis_index("s")
    wid = s * _NC + c
    ebase = wid * _EPW

    def _start_p(ci, slot):
        sl = pl.ds(ebase + ci * _K, _K)
        pltpu.async_copy(col_hbm.at[sl], cbuf.at[slot], psem.at[slot])
        pltpu.async_copy(row_hbm.at[sl], rbv.at[slot], psem.at[slot])
        pltpu.async_copy(val_hbm.at[sl], vbuf.at[slot], psem.at[slot])

    def _wait_p(ci, slot):
        sl = pl.ds(ebase + ci * _K, _K)
        pltpu.make_async_copy(col_hbm.at[sl], cbuf.at[slot],
                              psem.at[slot]).wait()
        pltpu.make_async_copy(row_hbm.at[sl], rbv.at[slot],
                              psem.at[slot]).wait()
        pltpu.make_async_copy(val_hbm.at[sl], vbuf.at[slot],
                              psem.at[slot]).wait()

    def _zrow(r, carry):
        for j in range(_D // 16):
            gbuf[2, r, pl.ds(j * 16, 16)] = jnp.zeros((16,), jnp.float32)
        return carry
    lax.fori_loop(0, _K, _zrow, 0)

    nz0, rem0 = _RPT0 // _K, _RPT0 % _K
    nzL, remL = _RPTL // _K, _RPTL % _K

    def _zero_copies(action):
        def _one(src, dst):
            d = pltpu.make_async_copy(src, dst, ssem.at[2])
            d.start() if action == "start" else d.wait()

        @pl.when(s < _NS - 1)
        def _():
            for q in range(nz0):
                _one(gbuf.at[2], acc.at[pl.ds(s * _RPT0 + q * _K, _K)])
            if rem0:
                _one(gbuf.at[2, pl.ds(0, rem0)],
                     acc.at[pl.ds(s * _RPT0 + _RPT0 - rem0, rem0)])

        @pl.when(s == _NS - 1)
        def _():
            for q in range(nzL):
                _one(gbuf.at[2], acc.at[pl.ds(s * _RPT0 + q * _K, _K)])
            if remL:
                _one(gbuf.at[2, pl.ds(0, remL)],
                     acc.at[pl.ds(s * _RPT0 + _RPTL - remL, remL)])

    _zero_copies("start")

    for j in range(_R):
        _start_p(j, j)
    for j in range(2):
        _wait_p(j, j)
        pltpu.async_copy(y_hbm.at[cbuf.at[j]], gbuf.at[j], gsem.at[j])

    _zero_copies("wait")
    plsc.subcore_barrier()

    def _multiply(b):
        def _grp(g, carry):
            sl16 = pl.ds(g * 16, 16)
            rbuf[b, sl16] = rbv[b, sl16]
            val16 = vbuf[b, sl16]
            for e in range(16):
                vsplat = _splat_lane(val16, e)
                row = g * 16 + e
                for j in range(_D // 16):
                    slj = pl.ds(j * 16, 16)
                    gbuf[b, row, slj] = gbuf[b, row, slj] * vsplat
            return carry
        lax.fori_loop(0, _K // 16, _grp, 0)

    def _chunk(ci, b, k=None, depth2=True):
        b2 = (b + 2) % _R
        pltpu.make_async_copy(y_hbm.at[cbuf.at[b]], gbuf.at[b],
                              gsem.at[b]).wait()
        _multiply(b)
        pltpu.async_copy(gbuf.at[b], acc.at[rbuf.at[b]], ssem.at[b],
                         add=True)
        if depth2:
            _start_p(jnp.minimum(ci + 3, _NCHUNK - 1), b)
        def _wait_s():
            pltpu.make_async_copy(gbuf.at[b2], acc.at[rbuf.at[b2]],
                                  ssem.at[b2]).wait()
        if k is None:
            _wait_s()
        else:
            pl.when(k >= 1)(_wait_s)
        if depth2:
            _wait_p(ci + 2, b2)
            pltpu.async_copy(y_hbm.at[cbuf.at[b2]], gbuf.at[b2],
                             gsem.at[b2])

    def _kbody(k, carry):
        base = 3 * k
        _chunk(base + 0, 0, k=k)
        _chunk(base + 1, 1)
        _chunk(base + 2, 2)
        return carry
    lax.fori_loop(0, 41, _kbody, 0)

    _chunk(_NCHUNK - 2, 0, depth2=False)
    _chunk(_NCHUNK - 1, 1, depth2=False)
    pltpu.make_async_copy(gbuf.at[1], acc.at[rbuf.at[1]], ssem.at[1]).wait()
    _wait_p(_NCHUNK - 1, 2)

    plsc.subcore_barrier()
    base = c * _N + s * _RPT0

    @pl.when(s < _NS - 1)
    def _():
        pltpu.sync_copy(acc.at[pl.ds(s * _RPT0, _RPT0)],
                        out_hbm.at[pl.ds(base, _RPT0)])

    @pl.when(s == _NS - 1)
    def _():
        pltpu.sync_copy(acc.at[pl.ds(s * _RPT0, _RPTL)],
                        out_hbm.at[pl.ds(base, _RPTL)])


def _sc_agg(y, col, row, val):
    mesh = plsc.VectorSubcoreMesh(core_axis_name="c", subcore_axis_name="s")
    fn = pl.kernel(
        _sc_agg_body,
        mesh=mesh,
        out_type=jax.ShapeDtypeStruct((_NC * _N, _D), jnp.float32),
        scratch_types=[
            pltpu.VMEM((_R, _K), jnp.int32),
            pltpu.VMEM((_R, _K), jnp.int32),
            pltpu.VMEM((_R, _K), jnp.float32),
            pltpu.VMEM((_R, _K), jnp.int32),
            pltpu.VMEM((_R, _K, _D), jnp.float32),
            pltpu.VMEM_SHARED((_N, _D), jnp.float32),
            pltpu.SemaphoreType.DMA((_R,)),
            pltpu.SemaphoreType.DMA((_R,)),
            pltpu.SemaphoreType.DMA((_R,)),
        ],
    )
    return fn(y, col, row, val)


def _combine_body(a_ref, b_ref, o_ref):
    o_ref[...] = a_ref[...] + b_ref[...]


def _combine(partials):
    bm = 1000
    nb = _N // bm
    return pl.pallas_call(
        _combine_body,
        grid=(nb,),
        in_specs=[
            pl.BlockSpec((bm, _D), lambda i: (i, 0)),
            pl.BlockSpec((bm, _D), lambda i: (i + nb, 0)),
        ],
        out_specs=pl.BlockSpec((bm, _D), lambda i: (i, 0)),
        out_shape=jax.ShapeDtypeStruct((_N, _D), jnp.float32),
    )(partials, partials)


def kernel(x, adj_indices, adj_values, W, b):
    val = adj_values.astype(jnp.float32)
    y, row, col = _linear(x, W, b, adj_indices.astype(jnp.int32))
    partials = _sc_agg(y, col, row, val)
    return _combine(partials)


# speedup vs baseline: 2.8476x; 1.0278x over previous
import jax
import jax.numpy as jnp
from jax import lax
from jax.experimental import pallas as pl
from jax.experimental.pallas import tpu as pltpu
from jax.experimental.pallas import tpu_sc as plsc

_N = 10000
_E = 320000
_D = 128
_NC = 2
_NS = 16
_NW = _NC * _NS
_EPW = _E // _NW
_K = 80
_NCHUNK = _EPW // _K
_R = 3
_RPT0 = 632
_RPTL = _N - (_NS - 1) * _RPT0


def _linear_body(x_ref, w_ref, b_ref, adj_ref, o_ref, row_ref, col_ref):
    o_ref[...] = lax.dot_general(
        x_ref[...], w_ref[...], (((1,), (1,)), ((), ())),
        preferred_element_type=jnp.float32) + b_ref[...]

    @pl.when(pl.program_id(0) == 0)
    def _():
        row_ref[...] = adj_ref[0, :]
        col_ref[...] = adj_ref[1, :]


def _linear(x, W, b, adj):
    bm = 2000
    return pl.pallas_call(
        _linear_body,
        grid=(_N // bm,),
        in_specs=[
            pl.BlockSpec((bm, _D), lambda i: (i, 0)),
            pl.BlockSpec((_D, _D), lambda i: (0, 0)),
            pl.BlockSpec((1, _D), lambda i: (0, 0)),
            pl.BlockSpec((2, _E), lambda i: (0, 0)),
        ],
        out_specs=[
            pl.BlockSpec((bm, _D), lambda i: (i, 0)),
            pl.BlockSpec((_E,), lambda i: (0,)),
            pl.BlockSpec((_E,), lambda i: (0,)),
        ],
        out_shape=[
            jax.ShapeDtypeStruct((_N, _D), jnp.float32),
            jax.ShapeDtypeStruct((_E,), jnp.int32),
            jax.ShapeDtypeStruct((_E,), jnp.int32),
        ],
    )(x, W, b.reshape(1, _D), adj)


def _splat_lane(vec16, lane):
    return lax.gather(
        vec16, jnp.full((16, 1), lane, jnp.int32),
        lax.GatherDimensionNumbers(
            offset_dims=(), collapsed_slice_dims=(0,), start_index_map=(0,)),
        slice_sizes=(1,),
        mode=lax.GatherScatterMode.PROMISE_IN_BOUNDS)


def _sc_agg_body(y_hbm, col_hbm, row_hbm, val_hbm, out_hbm,
                 cbuf, rbv, vbuf, rbuf, gbuf, acc, gsem, psem, ssem):
    c = lax.axis_index("c")
    s = lax.axis_index("s")
    wid = s * _NC + c
    ebase = wid * _EPW

    def _start_p(ci, slot):
        sl = pl.ds(ebase + ci * _K, _K)
        pltpu.async_copy(col_hbm.at[sl], cbuf.at[slot], psem.at[slot])
        pltpu.async_copy(row_hbm.at[sl], rbv.at[slot], psem.at[slot])
        pltpu.async_copy(val_hbm.at[sl], vbuf.at[slot], psem.at[slot])

    def _wait_p(ci, slot):
        sl = pl.ds(ebase + ci * _K, _K)
        pltpu.make_async_copy(col_hbm.at[sl], cbuf.at[slot],
                              psem.at[slot]).wait()
        pltpu.make_async_copy(row_hbm.at[sl], rbv.at[slot],
                              psem.at[slot]).wait()
        pltpu.make_async_copy(val_hbm.at[sl], vbuf.at[slot],
                              psem.at[slot]).wait()

    def _zrow(r, carry):
        for j in range(_D // 16):
            gbuf[2, r, pl.ds(j * 16, 16)] = jnp.zeros((16,), jnp.float32)
        return carry
    lax.fori_loop(0, _K, _zrow, 0)

    nz0, rem0 = _RPT0 // _K, _RPT0 % _K
    nzL, remL = _RPTL // _K, _RPTL % _K

    def _zero_copies(action):
        def _one(src, dst):
            d = pltpu.make_async_copy(src, dst, ssem.at[2])
            d.start() if action == "start" else d.wait()

        @pl.when(s < _NS - 1)
        def _():
            for q in range(nz0):
                _one(gbuf.at[2], acc.at[pl.ds(s * _RPT0 + q * _K, _K)])
            if rem0:
                _one(gbuf.at[2, pl.ds(0, rem0)],
                     acc.at[pl.ds(s * _RPT0 + _RPT0 - rem0, rem0)])

        @pl.when(s == _NS - 1)
        def _():
            for q in range(nzL):
                _one(gbuf.at[2], acc.at[pl.ds(s * _RPT0 + q * _K, _K)])
            if remL:
                _one(gbuf.at[2, pl.ds(0, remL)],
                     acc.at[pl.ds(s * _RPT0 + _RPTL - remL, remL)])

    _zero_copies("start")

    for j in range(_R):
        _start_p(j, j)
    for j in range(2):
        _wait_p(j, j)
        pltpu.async_copy(y_hbm.at[cbuf.at[j]], gbuf.at[j], gsem.at[j])

    _zero_copies("wait")
    plsc.subcore_barrier()

    def _multiply(b):
        def _grp(g, carry):
            sl16 = pl.ds(g * 16, 16)
            rbuf[b, sl16] = rbv[b, sl16]
            val16 = vbuf[b, sl16]
            for e in range(16):
                vsplat = _splat_lane(val16, e)
                row = g * 16 + e
                for j in range(_D // 16):
                    slj = pl.ds(j * 16, 16)
                    gbuf[b, row, slj] = gbuf[b, row, slj] * vsplat
            return carry
        lax.fori_loop(0, _K // 16, _grp, 0)

    def _chunk(ci, b, k=None, depth2=True):
        b2 = (b + 2) % _R
        pltpu.make_async_copy(y_hbm.at[cbuf.at[b]], gbuf.at[b],
                              gsem.at[b]).wait()
        _multiply(b)
        pltpu.async_copy(gbuf.at[b], acc.at[rbuf.at[b]], ssem.at[b],
                         add=True)
        if depth2:
            _start_p(jnp.minimum(ci + 3, _NCHUNK - 1), b)
        def _wait_s():
            pltpu.make_async_copy(gbuf.at[b2], acc.at[rbuf.at[b2]],
                                  ssem.at[b2]).wait()
        if k is None:
            _wait_s()
        else:
            pl.when(k >= 1)(_wait_s)
        if depth2:
            _wait_p(ci + 2, b2)
            pltpu.async_copy(y_hbm.at[cbuf.at[b2]], gbuf.at[b2],
                             gsem.at[b2])

    def _kbody(k, carry):
        base = 3 * k
        _chunk(base + 0, 0, k=k)
        _chunk(base + 1, 1)
        _chunk(base + 2, 2)
        return carry
    lax.fori_loop(0, 41, _kbody, 0)

    _chunk(_NCHUNK - 2, 0, depth2=False)
    _chunk(_NCHUNK - 1, 1, depth2=False)
    pltpu.make_async_copy(gbuf.at[1], acc.at[rbuf.at[1]], ssem.at[1]).wait()
    _wait_p(_NCHUNK - 1, 2)

    plsc.subcore_barrier()
    base = c * _N + s * _RPT0

    @pl.when(s < _NS - 1)
    def _():
        pltpu.sync_copy(acc.at[pl.ds(s * _RPT0, _RPT0)],
                        out_hbm.at[pl.ds(base, _RPT0)])

    @pl.when(s == _NS - 1)
    def _():
        pltpu.sync_copy(acc.at[pl.ds(s * _RPT0, _RPTL)],
                        out_hbm.at[pl.ds(base, _RPTL)])


def _sc_agg(y, col, row, val):
    mesh = plsc.VectorSubcoreMesh(core_axis_name="c", subcore_axis_name="s")
    fn = pl.kernel(
        _sc_agg_body,
        mesh=mesh,
        out_type=jax.ShapeDtypeStruct((_NC * _N, _D), jnp.float32),
        scratch_types=[
            pltpu.VMEM((_R, _K), jnp.int32),
            pltpu.VMEM((_R, _K), jnp.int32),
            pltpu.VMEM((_R, _K), jnp.float32),
            pltpu.VMEM((_R, _K), jnp.int32),
            pltpu.VMEM((_R, _K, _D), jnp.float32),
            pltpu.VMEM_SHARED((_N, _D), jnp.float32),
            pltpu.SemaphoreType.DMA((_R,)),
            pltpu.SemaphoreType.DMA((_R,)),
            pltpu.SemaphoreType.DMA((_R,)),
        ],
    )
    return fn(y, col, row, val)


def _combine_body(a_ref, b_ref, o_ref):
    o_ref[...] = a_ref[...] + b_ref[...]


def _combine(partials):
    bm = 2000
    nb = _N // bm
    return pl.pallas_call(
        _combine_body,
        grid=(nb,),
        in_specs=[
            pl.BlockSpec((bm, _D), lambda i: (i, 0)),
            pl.BlockSpec((bm, _D), lambda i: (i + nb, 0)),
        ],
        out_specs=pl.BlockSpec((bm, _D), lambda i: (i, 0)),
        out_shape=jax.ShapeDtypeStruct((_N, _D), jnp.float32),
    )(partials, partials)


def kernel(x, adj_indices, adj_values, W, b):
    val = adj_values.astype(jnp.float32)
    y, row, col = _linear(x, W, b, adj_indices.astype(jnp.int32))
    partials = _sc_agg(y, col, row, val)
    return _combine(partials)
